# Initial kernel scaffold; baseline (speedup 1.0000x reference)
import jax
import jax.numpy as jnp
from jax.experimental import pallas as pl


def kernel(x, edge_index, W1, b1, W2, b2, alpha):
    raise NotImplementedError("write your pallas kernel here")


# speedup vs baseline: 17.0247x; 17.0247x over previous
import functools

import jax
import jax.numpy as jnp
from jax import lax
from jax.experimental import pallas as pl
from jax.experimental.pallas import tpu as pltpu
from jax.experimental.pallas import tpu_sc as plsc

N = 10000
D = 128
E = 320000

NC = 2
NS = 16
NW = NC * NS
EPT = E // NW
CHUNK = 80
NCHUNK = EPT // CHUNK
N_PAD = 10240
RPT = N_PAD // NS
DEGW = 8

_mesh = plsc.VectorSubcoreMesh(core_axis_name="c", subcore_axis_name="s")


@functools.partial(
    pl.kernel,
    mesh=_mesh,
    out_type=jax.ShapeDtypeStruct((NC, NS, RPT, DEGW), jnp.float32),
    scratch_types=[
        pltpu.VMEM((NCHUNK, CHUNK), jnp.int32),
        pltpu.VMEM((CHUNK, DEGW), jnp.float32),
        pltpu.VMEM_SHARED((N_PAD, DEGW), jnp.float32),
    ],
)
def _deg_fn(dst_hbm, ones_hbm, zeros_hbm, cnt_hbm, dst_v, ones_v, acc_sh):
    c = lax.axis_index("c")
    s = lax.axis_index("s")
    wid = c * NS + s
    pltpu.sync_copy(dst_hbm.at[wid], dst_v)
    pltpu.sync_copy(ones_hbm, ones_v)
    pltpu.sync_copy(zeros_hbm, acc_sh.at[pl.ds(s * RPT, RPT), :])
    plsc.subcore_barrier()

    def body(j, carry):
        pltpu.sync_copy(ones_v, acc_sh.at[dst_v.at[j]], add=True)
        return carry

    lax.fori_loop(0, NCHUNK, body, 0)
    plsc.subcore_barrier()
    pltpu.sync_copy(acc_sh.at[pl.ds(s * RPT, RPT), :], cnt_hbm.at[c, s])


@functools.partial(
    pl.kernel,
    mesh=_mesh,
    out_type=jax.ShapeDtypeStruct((NC, NS, RPT, D), jnp.float32),
    scratch_types=[
        pltpu.VMEM((NCHUNK, CHUNK), jnp.int32),
        pltpu.VMEM((NCHUNK, CHUNK), jnp.int32),
        pltpu.VMEM((CHUNK, D), jnp.float32),
        pltpu.VMEM_SHARED((N_PAD, D), jnp.float32),
        pltpu.SemaphoreType.DMA,
    ],
)
def _scat_fn(g_hbm, src_hbm, dst_hbm, zeros_hbm, out_hbm,
             src_v, dst_v, rows_v, acc_sh, sem):
    c = lax.axis_index("c")
    s = lax.axis_index("s")
    wid = c * NS + s
    pltpu.sync_copy(src_hbm.at[wid], src_v)
    pltpu.sync_copy(dst_hbm.at[wid], dst_v)
    pltpu.sync_copy(zeros_hbm, acc_sh.at[pl.ds(s * RPT, RPT), :])
    plsc.subcore_barrier()

    def body(j, carry):
        pltpu.async_copy(g_hbm.at[src_v.at[j]], rows_v, sem).wait()
        pltpu.sync_copy(rows_v, acc_sh.at[dst_v.at[j]], add=True)
        return carry

    lax.fori_loop(0, NCHUNK, body, 0)
    plsc.subcore_barrier()
    pltpu.sync_copy(acc_sh.at[pl.ds(s * RPT, RPT), :], out_hbm.at[c, s])


BLK = 1000


def _tc1_body(cnt_ref, x_ref, w1_ref, g1_ref, dinv_ref):
    deg = cnt_ref[0, :, 0] + cnt_ref[1, :, 0] + 1.0
    dinv = lax.rsqrt(deg)
    h = jnp.dot(x_ref[...], w1_ref[...], preferred_element_type=jnp.float32)
    g1_ref[...] = h * dinv[:, None]
    dinv_ref[...] = dinv[:, None]


def _tc2_body(s0_ref, s1_ref, g1_ref, dinv_ref, w2_ref, b1_ref, g2_ref):
    z1 = (s0_ref[...] + s1_ref[...] + g1_ref[...]) * dinv_ref[...] + b1_ref[...]
    h1 = jnp.maximum(z1, 0.0)
    h2 = jnp.dot(h1, w2_ref[...], preferred_element_type=jnp.float32)
    g2_ref[...] = h2 * dinv_ref[...]


def _tc3_body(alpha_ref, s0_ref, s1_ref, g2_ref, dinv_ref, b2_ref, x_ref, out_ref):
    z2 = (s0_ref[...] + s1_ref[...] + g2_ref[...]) * dinv_ref[...] + b2_ref[...]
    a = jnp.clip(alpha_ref[0], -1.0, 1.0)
    y = x_ref[...] + a * z2
    nrm2 = jnp.sum(y * y, axis=1, keepdims=True)
    out_ref[...] = y * lax.rsqrt(jnp.maximum(nrm2, 1e-24))


def _row_spec(w):
    return pl.BlockSpec((BLK, w), lambda i: (i, 0))


def _full_spec(shape):
    return pl.BlockSpec(shape, lambda i: tuple(0 for _ in shape))


_tc1 = pl.pallas_call(
    _tc1_body,
    grid=(N // BLK,),
    in_specs=[
        pl.BlockSpec((2, BLK, 1), lambda i: (0, i, 0)),
        _row_spec(D),
        _full_spec((D, D)),
    ],
    out_specs=[_row_spec(D), _row_spec(1)],
    out_shape=[
        jax.ShapeDtypeStruct((N, D), jnp.float32),
        jax.ShapeDtypeStruct((N, 1), jnp.float32),
    ],
)

_tc2 = pl.pallas_call(
    _tc2_body,
    grid=(N // BLK,),
    in_specs=[
        _row_spec(D),
        _row_spec(D),
        _row_spec(D),
        _row_spec(1),
        _full_spec((D, D)),
        _full_spec((1, D)),
    ],
    out_specs=_row_spec(D),
    out_shape=jax.ShapeDtypeStruct((N, D), jnp.float32),
)

_tc3 = pl.pallas_call(
    _tc3_body,
    grid=(N // BLK,),
    in_specs=[
        pl.BlockSpec(memory_space=pltpu.SMEM),
        _row_spec(D),
        _row_spec(D),
        _row_spec(D),
        _row_spec(1),
        _full_spec((1, D)),
        _row_spec(D),
    ],
    out_specs=_row_spec(D),
    out_shape=jax.ShapeDtypeStruct((N, D), jnp.float32),
)


def _sc_partial_sums(g, src3, dst3, zeros_nd):
    parts = _scat_fn(g, src3, dst3, zeros_nd)
    parts = parts.reshape(NC, N_PAD, D)
    return parts[0, :N, :], parts[1, :N, :]


@jax.jit
def kernel(x, edge_index, W1, b1, W2, b2, alpha):
    src = edge_index[0].astype(jnp.int32)
    dst = edge_index[1].astype(jnp.int32)
    src3 = src.reshape(NW, NCHUNK, CHUNK)
    dst3 = dst.reshape(NW, NCHUNK, CHUNK)

    ones_deg = jnp.ones((CHUNK, DEGW), jnp.float32)
    zeros_deg = jnp.zeros((RPT, DEGW), jnp.float32)
    zeros_nd = jnp.zeros((RPT, D), jnp.float32)

    cnt_parts = _deg_fn(dst3, ones_deg, zeros_deg)
    cnt = cnt_parts.reshape(NC, N_PAD, DEGW)[:, :N, :1]

    g1, dinv = _tc1(cnt, x, W1)
    s1a, s1b = _sc_partial_sums(g1, src3, dst3, zeros_nd)
    g2 = _tc2(s1a, s1b, g1, dinv, W2, b1.reshape(1, D))
    s2a, s2b = _sc_partial_sums(g2, src3, dst3, zeros_nd)
    return _tc3(alpha.reshape(1), s2a, s2b, g2, dinv, b2.reshape(1, D), x)


# speedup vs baseline: 17.4369x; 1.0242x over previous
import functools

import jax
import jax.numpy as jnp
from jax import lax
from jax.experimental import pallas as pl
from jax.experimental.pallas import tpu as pltpu
from jax.experimental.pallas import tpu_sc as plsc

N = 10000
D = 128
E = 320000

NC = 2
NS = 16
NW = NC * NS
CHUNK = 120
NCHUNK = 84
EPT = NCHUNK * CHUNK
E_PAD = NW * EPT
N_PAD = 10112
RPT = N_PAD // NS
DEGW = 8

_mesh = plsc.VectorSubcoreMesh(core_axis_name="c", subcore_axis_name="s")


@functools.partial(
    pl.kernel,
    mesh=_mesh,
    out_type=jax.ShapeDtypeStruct((NC, N_PAD, DEGW), jnp.float32),
    scratch_types=[
        pltpu.VMEM((NCHUNK, 2, CHUNK), jnp.int32),
        pltpu.VMEM((CHUNK, DEGW), jnp.float32),
        pltpu.VMEM_SHARED((N_PAD, DEGW), jnp.float32),
        pltpu.SemaphoreType.DMA,
    ],
)
def _deg_fn(idx_hbm, ones_hbm, zeros_hbm, cnt_hbm, idx_v, ones_v, acc_sh, sem):
    c = lax.axis_index("c")
    s = lax.axis_index("s")
    wid = c * NS + s
    pltpu.sync_copy(idx_hbm.at[wid], idx_v)
    pltpu.sync_copy(ones_hbm, ones_v)
    pltpu.sync_copy(zeros_hbm, acc_sh.at[pl.ds(s * RPT, RPT), :])
    plsc.subcore_barrier()

    def fire(j):
        pltpu.async_copy(ones_v, acc_sh.at[idx_v.at[j, 1]], sem, add=True)

    def drain(j):
        pltpu.make_async_copy(ones_v, acc_sh.at[idx_v.at[j, 1]], sem).wait()

    fire(0)
    fire(1)

    def body(j, carry):
        fire(j + 2)
        drain(j)
        return carry

    lax.fori_loop(0, NCHUNK - 2, body, 0)
    drain(NCHUNK - 2)
    drain(NCHUNK - 1)
    plsc.subcore_barrier()
    pltpu.sync_copy(acc_sh.at[pl.ds(s * RPT, RPT), :], cnt_hbm.at[c, pl.ds(s * RPT, RPT), :])


@functools.partial(
    pl.kernel,
    mesh=_mesh,
    out_type=jax.ShapeDtypeStruct((NC, N_PAD, D), jnp.float32),
    scratch_types=[
        pltpu.VMEM((6, 2, CHUNK), jnp.int32),
        pltpu.VMEM((3, CHUNK, D), jnp.float32),
        pltpu.VMEM_SHARED((N_PAD, D), jnp.float32),
        pltpu.SemaphoreType.DMA,
        pltpu.SemaphoreType.DMA,
        pltpu.SemaphoreType.DMA,
        pltpu.SemaphoreType.DMA,
        pltpu.SemaphoreType.DMA,
        pltpu.SemaphoreType.DMA,
        pltpu.SemaphoreType.DMA,
        pltpu.SemaphoreType.DMA,
        pltpu.SemaphoreType.DMA,
        pltpu.SemaphoreType.DMA,
        pltpu.SemaphoreType.DMA,
        pltpu.SemaphoreType.DMA,
    ],
)
def _scat_fn(g_hbm, idx_hbm, zeros_hbm, out_hbm, idx_v, rows_v, acc_sh,
             sg0, sg1, sg2, ss0, ss1, ss2, si0, si1, si2, si3, si4, si5):
    c = lax.axis_index("c")
    s = lax.axis_index("s")
    wid = c * NS + s
    sgs = (sg0, sg1, sg2)
    sss = (ss0, ss1, ss2)
    sis = (si0, si1, si2, si3, si4, si5)

    def fire_i(j, m):
        pltpu.async_copy(idx_hbm.at[wid, j], idx_v.at[m], sis[m])

    def wait_i(j, m):
        pltpu.make_async_copy(idx_hbm.at[wid, j], idx_v.at[m], sis[m]).wait()

    def fire_g(m, b):
        pltpu.async_copy(g_hbm.at[idx_v.at[m, 0]], rows_v.at[b], sgs[b])

    def wait_g(m, b):
        pltpu.make_async_copy(g_hbm.at[idx_v.at[m, 0]], rows_v.at[b], sgs[b]).wait()

    def fire_s(m, b):
        pltpu.async_copy(rows_v.at[b], acc_sh.at[idx_v.at[m, 1]], sss[b], add=True)

    def wait_s(m, b):
        pltpu.make_async_copy(rows_v.at[b], acc_sh.at[idx_v.at[m, 1]], sss[b]).wait()

    for j in range(6):
        pltpu.sync_copy(idx_hbm.at[wid, j], idx_v.at[j])
    fire_g(0, 0)
    fire_g(1, 1)
    pltpu.sync_copy(zeros_hbm, acc_sh.at[pl.ds(s * RPT, RPT), :])
    plsc.subcore_barrier()

    def body(o, carry):
        for k in range(6):
            j = 6 * o + k

            wait_g(k % 6, k % 3)
            fire_s(k % 6, k % 3)

            @pl.when(j >= 1)
            def _():
                wait_s((k + 5) % 6, (k + 2) % 3)

            @pl.when(jnp.logical_and(j >= 1, j + 5 < NCHUNK))
            def _():
                fire_i(j + 5, (k + 5) % 6)

            @pl.when(jnp.logical_and(j + 2 >= 6, j + 2 < NCHUNK))
            def _():
                wait_i(j + 2, (k + 2) % 6)

            @pl.when(j + 2 < NCHUNK)
            def _():
                fire_g((k + 2) % 6, (k + 2) % 3)

        return carry

    lax.fori_loop(0, NCHUNK // 6, body, 0)
    wait_s((NCHUNK - 1) % 6, (NCHUNK - 1) % 3)
    plsc.subcore_barrier()
    pltpu.sync_copy(acc_sh.at[pl.ds(s * RPT, RPT), :], out_hbm.at[c, pl.ds(s * RPT, RPT), :])


BLK = 1000


def _tc1_body(cnt_ref, x_ref, w1_ref, g1_ref, dinv_ref):
    deg = cnt_ref[0, :, 0] + cnt_ref[1, :, 0] + 1.0
    dinv = lax.rsqrt(deg)
    h = jnp.dot(x_ref[...], w1_ref[...], preferred_element_type=jnp.float32)
    g1_ref[...] = h * dinv[:, None]
    dinv_ref[...] = dinv[:, None]


def _tc2_body(sp_ref, g1_ref, dinv_ref, w2_ref, b1_ref, g2_ref):
    z1 = (sp_ref[0] + sp_ref[1] + g1_ref[...]) * dinv_ref[...] + b1_ref[...]
    h1 = jnp.maximum(z1, 0.0)
    h2 = jnp.dot(h1, w2_ref[...], preferred_element_type=jnp.float32)
    g2_ref[...] = h2 * dinv_ref[...]


def _tc3_body(alpha_ref, sp_ref, g2_ref, dinv_ref, b2_ref, x_ref, out_ref):
    z2 = (sp_ref[0] + sp_ref[1] + g2_ref[...]) * dinv_ref[...] + b2_ref[...]
    a = jnp.clip(alpha_ref[0], -1.0, 1.0)
    y = x_ref[...] + a * z2
    nrm2 = jnp.sum(y * y, axis=1, keepdims=True)
    out_ref[...] = y * lax.rsqrt(jnp.maximum(nrm2, 1e-24))


def _row_spec(w):
    return pl.BlockSpec((BLK, w), lambda i: (i, 0))


def _pair_spec(w):
    return pl.BlockSpec((2, BLK, w), lambda i: (0, i, 0))


def _full_spec(shape):
    return pl.BlockSpec(shape, lambda i: tuple(0 for _ in shape))


_tc1 = pl.pallas_call(
    _tc1_body,
    grid=(N // BLK,),
    in_specs=[_pair_spec(DEGW), _row_spec(D), _full_spec((D, D))],
    out_specs=[_row_spec(D), _row_spec(1)],
    out_shape=[
        jax.ShapeDtypeStruct((N_PAD, D), jnp.float32),
        jax.ShapeDtypeStruct((N, 1), jnp.float32),
    ],
)

_tc2 = pl.pallas_call(
    _tc2_body,
    grid=(N // BLK,),
    in_specs=[
        _pair_spec(D),
        _row_spec(D),
        _row_spec(1),
        _full_spec((D, D)),
        _full_spec((1, D)),
    ],
    out_specs=_row_spec(D),
    out_shape=jax.ShapeDtypeStruct((N_PAD, D), jnp.float32),
)

_tc3 = pl.pallas_call(
    _tc3_body,
    grid=(N // BLK,),
    in_specs=[
        pl.BlockSpec(memory_space=pltpu.SMEM),
        _pair_spec(D),
        _row_spec(D),
        _row_spec(1),
        _full_spec((1, D)),
        _row_spec(D),
    ],
    out_specs=_row_spec(D),
    out_shape=jax.ShapeDtypeStruct((N, D), jnp.float32),
)


@jax.jit
def kernel(x, edge_index, W1, b1, W2, b2, alpha):
    ei = edge_index.astype(jnp.int32)
    pad = jnp.full((2, E_PAD - E), N_PAD - 1, jnp.int32)
    ei = jnp.concatenate([ei, pad], axis=1)
    idx = ei.reshape(2, NW, NCHUNK, CHUNK).transpose(1, 2, 0, 3)

    ones_deg = jnp.ones((CHUNK, DEGW), jnp.float32)
    zeros_deg = jnp.zeros((RPT, DEGW), jnp.float32)
    zeros_nd = jnp.zeros((RPT, D), jnp.float32)

    cnt = _deg_fn(idx, ones_deg, zeros_deg)

    g1, dinv = _tc1(cnt, x, W1)
    s1 = _scat_fn(g1, idx, zeros_nd)
    g2 = _tc2(s1, g1, dinv, W2, b1.reshape(1, D))
    s2 = _scat_fn(g2, idx, zeros_nd)
    return _tc3(alpha.reshape(1), s2, g2, dinv, b2.reshape(1, D), x)
